# Initial kernel scaffold; baseline (speedup 1.0000x reference)
import jax
import jax.numpy as jnp
from jax.experimental import pallas as pl


def kernel(y, s, emb1_w, emb2_w):
    raise NotImplementedError("write your pallas kernel here")


# speedup vs baseline: 3.8254x; 3.8254x over previous
import functools

import jax
import jax.numpy as jnp
from jax import lax
from jax.experimental import pallas as pl
from jax.experimental.pallas import tpu as pltpu
from jax.experimental.pallas import tpu_sc as plsc

_NC = 2
_NS = 16
_NW = _NC * _NS
_B = 16384
_CHUNK = _B // _NW
_STEPS = _CHUNK // 16

_mesh = plsc.VectorSubcoreMesh(core_axis_name="c", subcore_axis_name="s")


@functools.partial(
    pl.kernel,
    out_type=jax.ShapeDtypeStruct((_B * 4,), jnp.float32),
    mesh=_mesh,
    compiler_params=pltpu.CompilerParams(needs_layout_passes=False),
    scratch_types=[
        pltpu.VMEM((_CHUNK,), jnp.int32),
        pltpu.VMEM((_CHUNK,), jnp.int32),
        pltpu.VMEM((16,), jnp.float32),
        pltpu.VMEM((_CHUNK * 4,), jnp.float32),
    ],
)
def _encode(y_hbm, s_hbm, tab_hbm, out_hbm, y_v, s_v, t_v, o_v):
    wid = lax.axis_index("s") * _NC + lax.axis_index("c")
    base = wid * _CHUNK
    pltpu.sync_copy(tab_hbm, t_v)
    pltpu.sync_copy(y_hbm.at[pl.ds(base, _CHUNK)], y_v)
    pltpu.sync_copy(s_hbm.at[pl.ds(base, _CHUNK)], s_v)

    lanes = lax.iota(jnp.int32, 16)
    for j in range(_STEPS):
        y16 = y_v[pl.ds(j * 16, 16)]
        s16 = s_v[pl.ds(j * 16, 16)]
        y2 = y16 + y16
        s2 = s16 + s16
        a = plsc.load_gather(t_v, [y2])
        b = plsc.load_gather(t_v, [y2 + 1])
        c = plsc.load_gather(t_v, [s2 + 6])
        d = plsc.load_gather(t_v, [s2 + 7])
        p = (j * 16 + lanes) * 4
        plsc.store_scatter(o_v, [p], a)
        plsc.store_scatter(o_v, [p + 1], b)
        plsc.store_scatter(o_v, [p + 2], c)
        plsc.store_scatter(o_v, [p + 3], d)

    pltpu.sync_copy(o_v, out_hbm.at[pl.ds(base * 4, _CHUNK * 4)])


def kernel(y, s, emb1_w, emb2_w):
    tab = jnp.concatenate(
        [emb1_w.reshape(-1), emb2_w.reshape(-1), jnp.zeros((4,), jnp.float32)]
    )
    out = _encode(y, s, tab)
    return out.reshape(_B, 4)
